# baseline (speedup 1.0000x reference)
import jax
import jax.numpy as jnp
from jax.experimental import pallas as pl
from jax.experimental.pallas import tpu as pltpu

VOCAB = 1000
N_EMBED = 128
T = 50
B = 1024
BB = 8
TOK = BB * T
GRID = B // BB


def _body(idx_ref, tok_ref, pos_ref, w_ref, b_ref, out_ref, x_ref):
    def gath(i, carry):
        x_ref[i, :] = tok_ref[idx_ref[0, 0, i], :]
        return carry

    jax.lax.fori_loop(0, TOK, gath, 0, unroll=8)
    x = x_ref[...] + pos_ref[...]
    out_ref[...] = (
        jnp.dot(x, w_ref[...], preferred_element_type=jnp.float32) + b_ref[...]
    )


def kernel(idx, tok_table, pos_table, W, b):
    idx3 = idx.astype(jnp.int32).reshape(GRID, 1, TOK)
    pos_tiled = jnp.tile(pos_table, (BB, 1))
    b2 = b.reshape(1, VOCAB)

    out = pl.pallas_call(
        _body,
        grid=(GRID,),
        in_specs=[
            pl.BlockSpec((1, 1, TOK), lambda i: (i, 0, 0),
                         memory_space=pltpu.SMEM),
            pl.BlockSpec((VOCAB, N_EMBED), lambda i: (0, 0)),
            pl.BlockSpec((TOK, N_EMBED), lambda i: (0, 0)),
            pl.BlockSpec((N_EMBED, VOCAB), lambda i: (0, 0)),
            pl.BlockSpec((1, VOCAB), lambda i: (0, 0)),
        ],
        out_specs=pl.BlockSpec((TOK, VOCAB), lambda i: (i, 0)),
        out_shape=jax.ShapeDtypeStruct((B * T, VOCAB), jnp.float32),
        scratch_shapes=[pltpu.VMEM((TOK, N_EMBED), jnp.float32)],
    )(idx3, tok_table, pos_tiled, W, b2)
    return out.reshape(B, T, VOCAB)
